# baseline (speedup 1.0000x reference)
import jax
import jax.numpy as jnp
from jax import lax
from jax.experimental import pallas as pl
from jax.experimental.pallas import tpu as pltpu
from jax.experimental.pallas import tpu_sc as plsc

DIM = 32
L = 50
NC = 2
NS = 16
NW = NC * NS
BC = 16
CL = BC * L

ENT_ROWS = 1000001
SLAB = 512
NPER = 62
NSLAB = NPER * NW
ENT_PAD = NSLAB * SLAB
NTILE = ENT_PAD // 128
SLABW = SLAB * DIM

_GATHER_DNUMS = lax.GatherDimensionNumbers(
    offset_dims=(), collapsed_slice_dims=(0,), start_index_map=(0,))


def _perm(vec, idx):
    return lax.gather(vec, idx.reshape(16, 1), _GATHER_DNUMS, (1,),
                      mode=lax.GatherScatterMode.PROMISE_IN_BOUNDS)


def _bcast_lane(vec, lane_idx):
    return _perm(vec, jnp.full((16,), lane_idx, jnp.int32))


def _allmax(v):
    lane = lax.broadcasted_iota(jnp.int32, (16,), 0)
    for k in (1, 2, 4, 8):
        v = jnp.maximum(v, _perm(v, lane ^ k))
    return v


def _allsum(v):
    lane = lax.broadcasted_iota(jnp.int32, (16,), 0)
    for k in (1, 2, 4, 8):
        v = v + _perm(v, lane ^ k)
    return v


_SUBS = []
_off = 0
while _off < CL:
    _n = min(128, CL - _off)
    _SUBS.append((_off, _n))
    _off += _n


def _retile_body(src_hbm, out_hbm, slab0, slab1, row0, row1, semi, semo):
    wid = lax.axis_index("s") * NC + lax.axis_index("c")
    lane = lax.broadcasted_iota(jnp.int32, (16,), 0)
    dbase = (lane % 8) * 128 + (lane // 8) * 4096

    def fire_in(s, slab_v):
        for dg in range(4):
            pltpu.async_copy(
                src_hbm.at[pl.ds((dg * NTILE + s * 4) * 1024, 4096)],
                slab_v.at[pl.ds(dg * 4096, 4096)], semi)

    def wait_in(slab_v):
        pltpu.make_async_copy(src_hbm.at[pl.ds(0, SLABW)], slab_v,
                              semi).wait()

    perms = [(lane + k) & 15 for k in range(16)]

    def transpose(slab_v, row_v):
        def col_body(c0, _):
            jj = c0 // 8
            cbase = c0 * 16
            goff = dbase + (jj * 1024 + (cbase - jj * 128))
            soff = cbase * DIM + lane
            for k in range(16):
                p = perms[k]
                gidx = goff + p
                g0 = plsc.load_gather(slab_v, [gidx])
                g1 = plsc.load_gather(slab_v, [gidx + 8192])
                sidx = soff + p * DIM
                plsc.store_scatter(row_v, [sidx], g0)
                plsc.store_scatter(row_v, [sidx + 16], g1)
            return 0

        lax.fori_loop(0, SLAB // 16, col_body, 0)

    def fire_out(s, row_v):
        pltpu.async_copy(row_v, out_hbm.at[pl.ds(s * SLABW, SLABW)], semo)

    def wait_out(row_v):
        pltpu.make_async_copy(row_v, out_hbm.at[pl.ds(0, SLABW)],
                              semo).wait()

    fire_in(wid, slab0)

    def pair_body(i2, _):
        s_a = wid + (2 * i2) * NW
        s_b = s_a + NW
        fire_in(s_b, slab1)
        wait_in(slab0)

        @pl.when(i2 > 0)
        def _():
            wait_out(row0)

        transpose(slab0, row0)
        fire_out(s_a, row0)

        @pl.when(i2 < (NPER // 2 - 1))
        def _():
            fire_in(s_b + NW, slab0)

        wait_in(slab1)

        @pl.when(i2 > 0)
        def _():
            wait_out(row1)

        transpose(slab1, row1)
        fire_out(s_b, row1)
        return 0

    lax.fori_loop(0, NPER // 2, pair_body, 0)
    wait_out(row0)
    wait_out(row1)


def _net_body(dr_hbm, de_hbm, rel_hbm, pid_hbm, nid_hbm, ent_hbm, edge_hbm,
              relt_hbm, out_hbm, pos_hbm, neg_hbm,
              dr_v, de_v, rel_i, pid_v, nid_v,
              w_v, e_v, r_v, p_v, n_v, out_v, sem):
    B = rel_hbm.shape[0]
    rows_per_w = B // NW
    nchunk = rows_per_w // BC
    wid = lax.axis_index("s") * NC + lax.axis_index("c")
    wstart = wid * rows_per_w

    lane = lax.broadcasted_iota(jnp.int32, (16,), 0)
    neg_inf = jnp.float32(-jnp.inf)

    def chunk_body(ci, _):
        base = wstart + ci * BC
        fbase = base * L
        pltpu.sync_copy(dr_hbm.at[pl.ds(fbase, CL)], dr_v)
        pltpu.sync_copy(de_hbm.at[pl.ds(fbase, CL)], de_v)
        pltpu.sync_copy(rel_hbm.at[pl.ds(base, BC)], rel_i)
        pltpu.sync_copy(pid_hbm.at[pl.ds(base, BC)], pid_v)
        pltpu.sync_copy(nid_hbm.at[pl.ds(base, BC)], nid_v)
        descs = []
        for off, n in _SUBS:
            descs.append(pltpu.async_copy(
                edge_hbm.at[dr_v.at[pl.ds(off, n)]],
                w_v.at[pl.ds(off, n)], sem))
            descs.append(pltpu.async_copy(
                ent_hbm.at[de_v.at[pl.ds(off, n)]],
                e_v.at[pl.ds(off, n), :], sem))
        descs.append(pltpu.async_copy(relt_hbm.at[rel_i], r_v, sem))
        descs.append(pltpu.async_copy(ent_hbm.at[pid_v], p_v, sem))
        descs.append(pltpu.async_copy(ent_hbm.at[nid_v], n_v, sem))
        for d in descs:
            d.wait()

        def row_body(b, _):
            off = b * L
            c0 = w_v[pl.ds(off, 16)]
            c1 = w_v[pl.ds(off + 16, 16)]
            c2 = w_v[pl.ds(off + 32, 16)]
            c3 = w_v[pl.ds(off + 48, 16)]
            c3 = jnp.where(lane < (L - 48), c3, neg_inf)
            m = _allmax(jnp.maximum(jnp.maximum(c0, c1), jnp.maximum(c2, c3)))
            x0 = jnp.exp(c0 - m)
            x1 = jnp.exp(c1 - m)
            x2 = jnp.exp(c2 - m)
            x3 = jnp.exp(c3 - m)
            s = _allsum(x0 + x1 + x2 + x3)
            inv = jnp.float32(1.0) / s
            wch = (x0 * inv, x1 * inv, x2 * inv, x3 * inv)
            acc0 = r_v[b, pl.ds(0, 16)]
            acc1 = r_v[b, pl.ds(16, 16)]
            for l in range(L):
                wl = _bcast_lane(wch[l // 16], l % 16)
                acc0 = acc0 + wl * e_v[off + l, pl.ds(0, 16)]
                acc1 = acc1 + wl * e_v[off + l, pl.ds(16, 16)]
            out_v[b, pl.ds(0, 16)] = acc0
            out_v[b, pl.ds(16, 16)] = acc1
            return 0

        lax.fori_loop(0, BC, row_body, 0)

        pltpu.sync_copy(out_v, out_hbm.at[pl.ds(base, BC), :])
        pltpu.sync_copy(p_v, pos_hbm.at[pl.ds(base, BC), :])
        pltpu.sync_copy(n_v, neg_hbm.at[pl.ds(base, BC), :])
        return 0

    lax.fori_loop(0, nchunk, chunk_body, 0)


def kernel(data_r, data_e, rel, pos_id, neg_id, entity_table, edge_table,
           rel_table):
    B = data_e.shape[0]
    dr_flat = data_r.astype(jnp.int32).reshape(-1)
    de_flat = data_e.astype(jnp.int32).reshape(-1)
    rel = rel.astype(jnp.int32)
    pos_id = pos_id.astype(jnp.int32)
    neg_id = neg_id.astype(jnp.int32)
    edge1d = edge_table.reshape(-1)
    f32 = jnp.float32

    mesh = plsc.VectorSubcoreMesh(core_axis_name="c", subcore_axis_name="s")

    ent_t = jnp.transpose(entity_table)
    ent_p = jnp.pad(ent_t, ((0, 0), (0, ENT_PAD - ENT_ROWS)))
    ent_p1d = (ent_p.reshape(4, 8, NTILE, 128)
               .transpose(0, 2, 1, 3).reshape(-1))

    retile = pl.kernel(
        _retile_body,
        out_type=jax.ShapeDtypeStruct((ENT_PAD * DIM,), f32),
        mesh=mesh,
        scratch_types=[
            pltpu.VMEM((SLABW,), f32),
            pltpu.VMEM((SLABW,), f32),
            pltpu.VMEM((SLABW,), f32),
            pltpu.VMEM((SLABW,), f32),
            pltpu.SemaphoreType.DMA,
            pltpu.SemaphoreType.DMA,
        ],
        compiler_params=pltpu.CompilerParams(use_tc_tiling_on_sc=False,
                                             needs_layout_passes=False),
    )
    ent2d = retile(ent_p1d).reshape(ENT_PAD, DIM)

    run = pl.kernel(
        _net_body,
        out_type=(
            jax.ShapeDtypeStruct((B, DIM), f32),
            jax.ShapeDtypeStruct((B, DIM), f32),
            jax.ShapeDtypeStruct((B, DIM), f32),
        ),
        mesh=mesh,
        scratch_types=[
            pltpu.VMEM((CL,), jnp.int32),
            pltpu.VMEM((CL,), jnp.int32),
            pltpu.VMEM((BC,), jnp.int32),
            pltpu.VMEM((BC,), jnp.int32),
            pltpu.VMEM((BC,), jnp.int32),
            pltpu.VMEM((CL + 16,), f32),
            pltpu.VMEM((CL, DIM), f32),
            pltpu.VMEM((BC, DIM), f32),
            pltpu.VMEM((BC, DIM), f32),
            pltpu.VMEM((BC, DIM), f32),
            pltpu.VMEM((BC, DIM), f32),
            pltpu.SemaphoreType.DMA,
        ],
        compiler_params=pltpu.CompilerParams(use_tc_tiling_on_sc=False),
    )
    out_t, pos_out, neg_out = run(dr_flat, de_flat, rel, pos_id, neg_id,
                                  ent2d, edge1d, rel_table)
    return (out_t, pos_out, neg_out)


# speedup vs baseline: 1.0707x; 1.0707x over previous
import jax
import jax.numpy as jnp
from jax import lax
from jax.experimental import pallas as pl
from jax.experimental.pallas import tpu as pltpu
from jax.experimental.pallas import tpu_sc as plsc

DIM = 32
L = 50
NC = 2
NS = 16
NW = NC * NS
BC = 16
CL = BC * L

ENT_ROWS = 1000001
SLAB = 512
NPER = 62
NSLAB = NPER * NW
ENT_PAD = NSLAB * SLAB
NTILE = ENT_PAD // 128
SLABW = SLAB * DIM

_GATHER_DNUMS = lax.GatherDimensionNumbers(
    offset_dims=(), collapsed_slice_dims=(0,), start_index_map=(0,))


def _perm(vec, idx):
    return lax.gather(vec, idx.reshape(16, 1), _GATHER_DNUMS, (1,),
                      mode=lax.GatherScatterMode.PROMISE_IN_BOUNDS)


def _bcast_lane(vec, lane_idx):
    return _perm(vec, jnp.full((16,), lane_idx, jnp.int32))


def _allmax(v):
    lane = lax.broadcasted_iota(jnp.int32, (16,), 0)
    for k in (1, 2, 4, 8):
        v = jnp.maximum(v, _perm(v, lane ^ k))
    return v


def _allsum(v):
    lane = lax.broadcasted_iota(jnp.int32, (16,), 0)
    for k in (1, 2, 4, 8):
        v = v + _perm(v, lane ^ k)
    return v


_SUBS = []
_off = 0
while _off < CL:
    _n = min(128, CL - _off)
    _SUBS.append((_off, _n))
    _off += _n


def _retile_body(src_hbm, out_hbm, slab0, slab1, row0, row1, semi, semo):
    wid = lax.axis_index("s") * NC + lax.axis_index("c")
    lane = lax.broadcasted_iota(jnp.int32, (16,), 0)
    dbase = (lane % 8) * 128 + (lane // 8) * 4096

    def fire_in(s, slab_v):
        for dg in range(4):
            pltpu.async_copy(
                src_hbm.at[pl.ds((dg * NTILE + s * 4) * 1024, 4096)],
                slab_v.at[pl.ds(dg * 4096, 4096)], semi)

    def wait_in(slab_v):
        pltpu.make_async_copy(src_hbm.at[pl.ds(0, SLABW)], slab_v,
                              semi).wait()

    perms = [(lane + k) & 15 for k in range(16)]

    def transpose(slab_v, row_v):
        def col_body(c0, _):
            jj = c0 // 8
            cbase = c0 * 16
            goff = dbase + (jj * 1024 + (cbase - jj * 128))
            soff = cbase * DIM + lane
            for k in range(16):
                p = perms[k]
                gidx = goff + p
                g0 = plsc.load_gather(slab_v, [gidx])
                g1 = plsc.load_gather(slab_v, [gidx + 8192])
                sidx = soff + p * DIM
                plsc.store_scatter(row_v, [sidx], g0)
                plsc.store_scatter(row_v, [sidx + 16], g1)
            return 0

        lax.fori_loop(0, SLAB // 16, col_body, 0)

    def fire_out(s, row_v):
        pltpu.async_copy(row_v, out_hbm.at[pl.ds(s * SLABW, SLABW)], semo)

    def wait_out(row_v):
        pltpu.make_async_copy(row_v, out_hbm.at[pl.ds(0, SLABW)],
                              semo).wait()

    fire_in(wid, slab0)

    def pair_body(i2, _):
        s_a = wid + (2 * i2) * NW
        s_b = s_a + NW
        fire_in(s_b, slab1)
        wait_in(slab0)

        @pl.when(i2 > 0)
        def _():
            wait_out(row0)

        transpose(slab0, row0)
        fire_out(s_a, row0)

        @pl.when(i2 < (NPER // 2 - 1))
        def _():
            fire_in(s_b + NW, slab0)

        wait_in(slab1)

        @pl.when(i2 > 0)
        def _():
            wait_out(row1)

        transpose(slab1, row1)
        fire_out(s_b, row1)
        return 0

    lax.fori_loop(0, NPER // 2, pair_body, 0)
    wait_out(row0)
    wait_out(row1)


def _net_body(dr_hbm, de_hbm, rel_hbm, pid_hbm, nid_hbm, ent_hbm, edge_hbm,
              relt_hbm, out_hbm, pos_hbm, neg_hbm, *scr):
    B = rel_hbm.shape[0]
    rows_per_w = B // NW
    nchunk = rows_per_w // BC
    wid = lax.axis_index("s") * NC + lax.axis_index("c")
    wstart = wid * rows_per_w

    bufs = (scr[0:11], scr[11:22])
    sem_s, sem_g, sem_o = scr[22:25]
    lane = lax.broadcasted_iota(jnp.int32, (16,), 0)
    neg_inf = jnp.float32(-jnp.inf)

    def stage(ci):
        dr_v, de_v, rel_i, pid_v, nid_v = bufs[ci % 2][:5]
        base = wstart + ci * BC
        fbase = base * L
        return [
            pltpu.async_copy(dr_hbm.at[pl.ds(fbase, CL)], dr_v, sem_s),
            pltpu.async_copy(de_hbm.at[pl.ds(fbase, CL)], de_v, sem_s),
            pltpu.async_copy(rel_hbm.at[pl.ds(base, BC)], rel_i, sem_s),
            pltpu.async_copy(pid_hbm.at[pl.ds(base, BC)], pid_v, sem_s),
            pltpu.async_copy(nid_hbm.at[pl.ds(base, BC)], nid_v, sem_s),
        ]

    def fire_gathers(ci):
        dr_v, de_v, rel_i, pid_v, nid_v, w_v, e_v, r_v, p_v, n_v, _ = \
            bufs[ci % 2]
        descs = []
        for off, n in _SUBS:
            descs.append(pltpu.async_copy(
                edge_hbm.at[dr_v.at[pl.ds(off, n)]],
                w_v.at[pl.ds(off, n)], sem_g))
            descs.append(pltpu.async_copy(
                ent_hbm.at[de_v.at[pl.ds(off, n)]],
                e_v.at[pl.ds(off, n), :], sem_g))
        descs.append(pltpu.async_copy(relt_hbm.at[rel_i], r_v, sem_g))
        descs.append(pltpu.async_copy(ent_hbm.at[pid_v], p_v, sem_g))
        descs.append(pltpu.async_copy(ent_hbm.at[nid_v], n_v, sem_g))
        return descs

    def compute(ci):
        w_v, e_v, r_v = bufs[ci % 2][5:8]
        out_v = bufs[ci % 2][10]

        def row_body(b, _):
            off = b * L
            c0 = w_v[pl.ds(off, 16)]
            c1 = w_v[pl.ds(off + 16, 16)]
            c2 = w_v[pl.ds(off + 32, 16)]
            c3 = w_v[pl.ds(off + 48, 16)]
            c3 = jnp.where(lane < (L - 48), c3, neg_inf)
            m = _allmax(jnp.maximum(jnp.maximum(c0, c1), jnp.maximum(c2, c3)))
            x0 = jnp.exp(c0 - m)
            x1 = jnp.exp(c1 - m)
            x2 = jnp.exp(c2 - m)
            x3 = jnp.exp(c3 - m)
            s = _allsum(x0 + x1 + x2 + x3)
            inv = jnp.float32(1.0) / s
            wch = (x0 * inv, x1 * inv, x2 * inv, x3 * inv)
            acc0 = r_v[b, pl.ds(0, 16)]
            acc1 = r_v[b, pl.ds(16, 16)]
            for l in range(L):
                wl = _bcast_lane(wch[l // 16], l % 16)
                acc0 = acc0 + wl * e_v[off + l, pl.ds(0, 16)]
                acc1 = acc1 + wl * e_v[off + l, pl.ds(16, 16)]
            out_v[b, pl.ds(0, 16)] = acc0
            out_v[b, pl.ds(16, 16)] = acc1
            return 0

        lax.fori_loop(0, BC, row_body, 0)

    def fire_out(ci):
        p_v, n_v, out_v = bufs[ci % 2][8:11]
        base = wstart + ci * BC
        return [
            pltpu.async_copy(out_v, out_hbm.at[pl.ds(base, BC), :], sem_o),
            pltpu.async_copy(p_v, pos_hbm.at[pl.ds(base, BC), :], sem_o),
            pltpu.async_copy(n_v, neg_hbm.at[pl.ds(base, BC), :], sem_o),
        ]

    st = {0: stage(0)}
    for d in st[0]:
        d.wait()
    gd = {0: fire_gathers(0)}
    st[1] = stage(1)
    od = {}
    for ci in range(nchunk):
        if ci + 1 < nchunk:
            for d in st[ci + 1]:
                d.wait()
            gd[ci + 1] = fire_gathers(ci + 1)
        if ci + 2 < nchunk:
            st[ci + 2] = stage(ci + 2)
        for d in gd[ci]:
            d.wait()
        if ci >= 2:
            for d in od[ci - 2]:
                d.wait()
        compute(ci)
        od[ci] = fire_out(ci)
    for d in od[nchunk - 2] + od[nchunk - 1]:
        d.wait()


def kernel(data_r, data_e, rel, pos_id, neg_id, entity_table, edge_table,
           rel_table):
    B = data_e.shape[0]
    dr_flat = data_r.astype(jnp.int32).reshape(-1)
    de_flat = data_e.astype(jnp.int32).reshape(-1)
    rel = rel.astype(jnp.int32)
    pos_id = pos_id.astype(jnp.int32)
    neg_id = neg_id.astype(jnp.int32)
    edge1d = edge_table.reshape(-1)
    f32 = jnp.float32

    mesh = plsc.VectorSubcoreMesh(core_axis_name="c", subcore_axis_name="s")

    ent_t = jnp.transpose(entity_table)
    ent_p = jnp.pad(ent_t, ((0, 0), (0, ENT_PAD - ENT_ROWS)))
    ent_p1d = (ent_p.reshape(4, 8, NTILE, 128)
               .transpose(0, 2, 1, 3).reshape(-1))

    retile = pl.kernel(
        _retile_body,
        out_type=jax.ShapeDtypeStruct((ENT_PAD * DIM,), f32),
        mesh=mesh,
        scratch_types=[
            pltpu.VMEM((SLABW,), f32),
            pltpu.VMEM((SLABW,), f32),
            pltpu.VMEM((SLABW,), f32),
            pltpu.VMEM((SLABW,), f32),
            pltpu.SemaphoreType.DMA,
            pltpu.SemaphoreType.DMA,
        ],
        compiler_params=pltpu.CompilerParams(use_tc_tiling_on_sc=False,
                                             needs_layout_passes=False),
    )
    ent2d = retile(ent_p1d).reshape(ENT_PAD, DIM)

    run = pl.kernel(
        _net_body,
        out_type=(
            jax.ShapeDtypeStruct((B, DIM), f32),
            jax.ShapeDtypeStruct((B, DIM), f32),
            jax.ShapeDtypeStruct((B, DIM), f32),
        ),
        mesh=mesh,
        scratch_types=[
            pltpu.VMEM((CL,), jnp.int32),
            pltpu.VMEM((CL,), jnp.int32),
            pltpu.VMEM((BC,), jnp.int32),
            pltpu.VMEM((BC,), jnp.int32),
            pltpu.VMEM((BC,), jnp.int32),
            pltpu.VMEM((CL + 16,), f32),
            pltpu.VMEM((CL, DIM), f32),
            pltpu.VMEM((BC, DIM), f32),
            pltpu.VMEM((BC, DIM), f32),
            pltpu.VMEM((BC, DIM), f32),
            pltpu.VMEM((BC, DIM), f32),
        ] * 2 + [
            pltpu.SemaphoreType.DMA,
            pltpu.SemaphoreType.DMA,
            pltpu.SemaphoreType.DMA,
        ],
        compiler_params=pltpu.CompilerParams(use_tc_tiling_on_sc=False),
    )
    out_t, pos_out, neg_out = run(dr_flat, de_flat, rel, pos_id, neg_id,
                                  ent2d, edge1d, rel_table)
    return (out_t, pos_out, neg_out)
